# Initial kernel scaffold; baseline (speedup 1.0000x reference)
import jax
import jax.numpy as jnp
from jax.experimental import pallas as pl


def kernel(s_buf, a_buf, r_buf, s_next_buf, dw_buf, ind):
    raise NotImplementedError("write your pallas kernel here")


# speedup vs baseline: 1.1246x; 1.1246x over previous
import functools

import jax
import jax.numpy as jnp
from jax import lax
from jax.experimental import pallas as pl
from jax.experimental.pallas import tpu as pltpu
from jax.experimental.pallas import tpu_sc as plsc


@functools.lru_cache(maxsize=None)
def _make_sample_kernel(M, D, B):
    info = plsc.get_sparse_core_info()
    num_cores = info.num_cores
    num_subcores = info.num_subcores
    nw = num_cores * num_subcores
    b_per_w = B // nw
    assert b_per_w * nw == B and b_per_w % 8 == 0

    mesh = plsc.VectorSubcoreMesh(core_axis_name="c", subcore_axis_name="s")

    @functools.partial(
        pl.kernel,
        mesh=mesh,
        out_type=(
            jax.ShapeDtypeStruct((B, D), jnp.float32),
            jax.ShapeDtypeStruct((B,), jnp.int32),
            jax.ShapeDtypeStruct((B,), jnp.float32),
            jax.ShapeDtypeStruct((B, D), jnp.float32),
            jax.ShapeDtypeStruct((B,), jnp.int32),
        ),
        scratch_types=[
            pltpu.VMEM((b_per_w,), jnp.int32),
            pltpu.VMEM((b_per_w, D), jnp.float32),
            pltpu.VMEM((b_per_w,), jnp.int32),
            pltpu.VMEM((b_per_w,), jnp.float32),
            pltpu.VMEM((b_per_w, D), jnp.float32),
            pltpu.VMEM((b_per_w,), jnp.int32),
            pltpu.SemaphoreType.DMA,
            pltpu.SemaphoreType.DMA,
            pltpu.SemaphoreType.DMA,
            pltpu.SemaphoreType.DMA,
            pltpu.SemaphoreType.DMA,
        ],
    )
    def sample_kernel(
        s_hbm, a_hbm, r_hbm, sn_hbm, dw_hbm, idx_hbm,
        s_out, a_out, r_out, sn_out, dw_out,
        idx_v, s_v, a_v, r_v, sn_v, dw_v,
        sem_s, sem_a, sem_r, sem_sn, sem_dw,
    ):
        wid = lax.axis_index("s") * num_cores + lax.axis_index("c")
        base = wid * b_per_w
        sl = pl.ds(base, b_per_w)

        pltpu.sync_copy(idx_hbm.at[sl], idx_v)

        cp_s = pltpu.async_copy(s_hbm.at[idx_v], s_v, sem_s)
        cp_sn = pltpu.async_copy(sn_hbm.at[idx_v], sn_v, sem_sn)
        cp_a = pltpu.async_copy(a_hbm.at[idx_v], a_v, sem_a)
        cp_r = pltpu.async_copy(r_hbm.at[idx_v], r_v, sem_r)
        cp_dw = pltpu.async_copy(dw_hbm.at[idx_v], dw_v, sem_dw)

        cp_s.wait()
        pltpu.sync_copy(s_v, s_out.at[sl])
        cp_sn.wait()
        pltpu.sync_copy(sn_v, sn_out.at[sl])
        cp_a.wait()
        pltpu.sync_copy(a_v, a_out.at[sl])
        cp_r.wait()
        pltpu.sync_copy(r_v, r_out.at[sl])
        cp_dw.wait()
        pltpu.sync_copy(dw_v, dw_out.at[sl])

    return sample_kernel


def kernel(s_buf, a_buf, r_buf, s_next_buf, dw_buf, ind):
    M, D = s_buf.shape
    B = ind.shape[0]
    fn = _make_sample_kernel(M, D, B)
    s, a, r, s_next, dw = fn(
        s_buf,
        a_buf.reshape(M),
        r_buf.reshape(M),
        s_next_buf,
        dw_buf.reshape(M),
        ind,
    )
    return (s, a.reshape(B, 1), r.reshape(B, 1), s_next, dw.reshape(B, 1))


# speedup vs baseline: 5.1002x; 4.5352x over previous
import functools

import jax
import jax.numpy as jnp
from jax import lax
from jax.experimental import pallas as pl
from jax.experimental.pallas import tpu as pltpu
from jax.experimental.pallas import tpu_sc as plsc

_LANES = 16


@functools.lru_cache(maxsize=None)
def _make_sample_kernel(M, D, B):
    info = plsc.get_sparse_core_info()
    num_cores = info.num_cores
    num_subcores = info.num_subcores
    nw = num_cores * num_subcores
    b_per_w = B // nw
    assert b_per_w * nw == B and b_per_w % _LANES == 0
    n_grp = b_per_w // _LANES

    mesh = plsc.VectorSubcoreMesh(core_axis_name="c", subcore_axis_name="s")

    @functools.partial(
        pl.kernel,
        mesh=mesh,
        compiler_params=pltpu.CompilerParams(needs_layout_passes=False),
        out_type=(
            jax.ShapeDtypeStruct((B, D), jnp.float32),
            jax.ShapeDtypeStruct((B,), jnp.int32),
            jax.ShapeDtypeStruct((B,), jnp.float32),
            jax.ShapeDtypeStruct((B, D), jnp.float32),
            jax.ShapeDtypeStruct((B,), jnp.int32),
        ),
        scratch_types=[
            pltpu.VMEM((b_per_w,), jnp.int32),
            pltpu.VMEM((b_per_w, D), jnp.float32),
            pltpu.VMEM((b_per_w, D), jnp.float32),
            pltpu.VMEM((b_per_w, 128), jnp.int32),
            pltpu.VMEM((b_per_w, 128), jnp.float32),
            pltpu.VMEM((b_per_w, 128), jnp.int32),
            pltpu.VMEM((b_per_w,), jnp.int32),
            pltpu.VMEM((b_per_w,), jnp.float32),
            pltpu.VMEM((b_per_w,), jnp.int32),
            pltpu.SemaphoreType.DMA,
            pltpu.SemaphoreType.DMA,
            pltpu.SemaphoreType.DMA,
            pltpu.SemaphoreType.DMA,
            pltpu.SemaphoreType.DMA,
        ],
    )
    def sample_kernel(
        s_hbm, a_hbm, r_hbm, sn_hbm, dw_hbm, idx_hbm,
        s_out, a_out, r_out, sn_out, dw_out,
        idx_v, s_v, sn_v, a_win, r_win, dw_win,
        a_out_v, r_out_v, dw_out_v,
        sem_s, sem_sn, sem_a, sem_r, sem_dw,
    ):
        wid = lax.axis_index("s") * num_cores + lax.axis_index("c")
        base = wid * b_per_w
        sl = pl.ds(base, b_per_w)

        pltpu.sync_copy(idx_hbm.at[sl], idx_v)

        cp_s = pltpu.async_copy(s_hbm.at[idx_v], s_v, sem_s)
        cp_sn = pltpu.async_copy(sn_hbm.at[idx_v], sn_v, sem_sn)

        cps_a, cps_r, cps_dw = [], [], []
        for g in range(n_grp):
            v = idx_v[pl.ds(g * _LANES, _LANES)]
            for l in range(_LANES):
                e = pl.multiple_of((v[l] >> 7) << 7, 128)
                j = pl.ds(g * _LANES + l, 1)
                cps_a.append(pltpu.async_copy(
                    a_hbm.at[:, pl.ds(e, 128)], a_win.at[j, :], sem_a))
                cps_r.append(pltpu.async_copy(
                    r_hbm.at[:, pl.ds(e, 128)], r_win.at[j, :], sem_r))
                cps_dw.append(pltpu.async_copy(
                    dw_hbm.at[:, pl.ds(e, 128)], dw_win.at[j, :], sem_dw))

        for cp in cps_a:
            cp.wait()
        for cp in cps_r:
            cp.wait()
        for cp in cps_dw:
            cp.wait()

        for g in range(n_grp):
            gs = pl.ds(g * _LANES, _LANES)
            rows = lax.iota(jnp.int32, _LANES) + (g * _LANES)
            cols = idx_v[gs] & 127
            a_out_v[gs] = plsc.load_gather(a_win, [rows, cols])
            r_out_v[gs] = plsc.load_gather(r_win, [rows, cols])
            dw_out_v[gs] = plsc.load_gather(dw_win, [rows, cols])

        pltpu.sync_copy(a_out_v, a_out.at[sl])
        pltpu.sync_copy(r_out_v, r_out.at[sl])
        pltpu.sync_copy(dw_out_v, dw_out.at[sl])

        cp_s.wait()
        pltpu.sync_copy(s_v, s_out.at[sl])
        cp_sn.wait()
        pltpu.sync_copy(sn_v, sn_out.at[sl])

    return sample_kernel


def kernel(s_buf, a_buf, r_buf, s_next_buf, dw_buf, ind):
    M, D = s_buf.shape
    B = ind.shape[0]
    fn = _make_sample_kernel(M, D, B)
    s, a, r, s_next, dw = fn(
        s_buf, a_buf.T, r_buf.T, s_next_buf, dw_buf.T, ind
    )
    return (s, a.reshape(B, 1), r.reshape(B, 1), s_next, dw.reshape(B, 1))


# speedup vs baseline: 5.2701x; 1.0333x over previous
import functools

import jax
import jax.numpy as jnp
from jax import lax
from jax.experimental import pallas as pl
from jax.experimental.pallas import tpu as pltpu
from jax.experimental.pallas import tpu_sc as plsc

_LANES = 16


@functools.lru_cache(maxsize=None)
def _make_sample_kernel(M, D, B):
    info = plsc.get_sparse_core_info()
    num_cores = info.num_cores
    num_subcores = info.num_subcores
    nw = num_cores * num_subcores
    b_per_w = B // nw
    assert b_per_w * nw == B and b_per_w % _LANES == 0
    n_grp = b_per_w // _LANES

    mesh = plsc.VectorSubcoreMesh(core_axis_name="c", subcore_axis_name="s")

    @functools.partial(
        pl.kernel,
        mesh=mesh,
        compiler_params=pltpu.CompilerParams(needs_layout_passes=False),
        out_type=(
            jax.ShapeDtypeStruct((B, D), jnp.float32),
            jax.ShapeDtypeStruct((B,), jnp.int32),
            jax.ShapeDtypeStruct((B,), jnp.float32),
            jax.ShapeDtypeStruct((B, D), jnp.float32),
            jax.ShapeDtypeStruct((B,), jnp.int32),
        ),
        scratch_types=[
            pltpu.VMEM((b_per_w,), jnp.int32),
            pltpu.VMEM((b_per_w, D), jnp.float32),
            pltpu.VMEM((b_per_w, D), jnp.float32),
            pltpu.VMEM((_LANES, 128), jnp.int32),
            pltpu.VMEM((_LANES, 128), jnp.float32),
            pltpu.VMEM((_LANES, 128), jnp.int32),
            pltpu.VMEM((b_per_w,), jnp.int32),
            pltpu.VMEM((b_per_w,), jnp.float32),
            pltpu.VMEM((b_per_w,), jnp.int32),
            pltpu.SemaphoreType.DMA,
            pltpu.SemaphoreType.DMA,
            pltpu.SemaphoreType.DMA,
        ],
    )
    def sample_kernel(
        s_hbm, a_hbm, r_hbm, sn_hbm, dw_hbm, idx_hbm,
        s_out, a_out, r_out, sn_out, dw_out,
        idx_v, s_v, sn_v, a_win, r_win, dw_win,
        a_out_v, r_out_v, dw_out_v,
        sem_s, sem_sn, sem_w,
    ):
        wid = lax.axis_index("s") * num_cores + lax.axis_index("c")
        base = wid * b_per_w
        sl = pl.ds(base, b_per_w)

        pltpu.sync_copy(idx_hbm.at[sl], idx_v)

        cp_s = pltpu.async_copy(s_hbm.at[idx_v], s_v, sem_s)
        cp_sn = pltpu.async_copy(sn_hbm.at[idx_v], sn_v, sem_sn)

        def body(g, carry):
            gbase = pl.multiple_of(g * _LANES, _LANES)
            gs = pl.ds(gbase, _LANES)
            v = idx_v[gs]
            cps = []
            for l in range(_LANES):
                e = pl.multiple_of((v[l] >> 7) << 7, 128)
                jl = pl.ds(l, 1)
                cps.append(pltpu.async_copy(
                    a_hbm.at[:, pl.ds(e, 128)], a_win.at[jl, :], sem_w))
                cps.append(pltpu.async_copy(
                    r_hbm.at[:, pl.ds(e, 128)], r_win.at[jl, :], sem_w))
                cps.append(pltpu.async_copy(
                    dw_hbm.at[:, pl.ds(e, 128)], dw_win.at[jl, :], sem_w))
            for cp in cps:
                cp.wait()
            rows = lax.iota(jnp.int32, _LANES)
            cols = v & 127
            a_out_v[gs] = plsc.load_gather(a_win, [rows, cols])
            r_out_v[gs] = plsc.load_gather(r_win, [rows, cols])
            dw_out_v[gs] = plsc.load_gather(dw_win, [rows, cols])
            return carry

        lax.fori_loop(0, n_grp, body, 0)

        pltpu.sync_copy(a_out_v, a_out.at[sl])
        pltpu.sync_copy(r_out_v, r_out.at[sl])
        pltpu.sync_copy(dw_out_v, dw_out.at[sl])

        cp_s.wait()
        pltpu.sync_copy(s_v, s_out.at[sl])
        cp_sn.wait()
        pltpu.sync_copy(sn_v, sn_out.at[sl])

    return sample_kernel


def kernel(s_buf, a_buf, r_buf, s_next_buf, dw_buf, ind):
    M, D = s_buf.shape
    B = ind.shape[0]
    fn = _make_sample_kernel(M, D, B)
    s, a, r, s_next, dw = fn(
        s_buf, a_buf.T, r_buf.T, s_next_buf, dw_buf.T, ind
    )
    return (s, a.reshape(B, 1), r.reshape(B, 1), s_next, dw.reshape(B, 1))


# speedup vs baseline: 6.1405x; 1.1652x over previous
import functools

import jax
import jax.numpy as jnp
from jax import lax
from jax.experimental import pallas as pl
from jax.experimental.pallas import tpu as pltpu
from jax.experimental.pallas import tpu_sc as plsc

_LANES = 16


@functools.lru_cache(maxsize=None)
def _make_sample_kernel(M, D, B):
    info = plsc.get_sparse_core_info()
    num_cores = info.num_cores
    num_subcores = info.num_subcores
    nw = num_cores * num_subcores
    b_per_w = B // nw
    assert b_per_w * nw == B and b_per_w % _LANES == 0
    n_grp = b_per_w // _LANES

    mesh = plsc.VectorSubcoreMesh(core_axis_name="c", subcore_axis_name="s")

    @functools.partial(
        pl.kernel,
        mesh=mesh,
        compiler_params=pltpu.CompilerParams(needs_layout_passes=False),
        out_type=(
            jax.ShapeDtypeStruct((B, D), jnp.float32),
            jax.ShapeDtypeStruct((B,), jnp.int32),
            jax.ShapeDtypeStruct((B,), jnp.float32),
            jax.ShapeDtypeStruct((B, D), jnp.float32),
            jax.ShapeDtypeStruct((B,), jnp.int32),
        ),
        scratch_types=[
            pltpu.VMEM((b_per_w,), jnp.int32),
            pltpu.VMEM((b_per_w, D), jnp.float32),
            pltpu.VMEM((b_per_w, D), jnp.float32),
            pltpu.VMEM((b_per_w, 128), jnp.int32),
            pltpu.VMEM((b_per_w, 128), jnp.float32),
            pltpu.VMEM((b_per_w, 128), jnp.int32),
            pltpu.VMEM((b_per_w,), jnp.int32),
            pltpu.VMEM((b_per_w,), jnp.float32),
            pltpu.VMEM((b_per_w,), jnp.int32),
            pltpu.SemaphoreType.DMA,
            pltpu.SemaphoreType.DMA,
            pltpu.SemaphoreType.DMA((8,)),
        ],
    )
    def sample_kernel(
        s_hbm, a_hbm, r_hbm, sn_hbm, dw_hbm, idx_hbm,
        s_out, a_out, r_out, sn_out, dw_out,
        idx_v, s_v, sn_v, a_win, r_win, dw_win,
        a_out_v, r_out_v, dw_out_v,
        sem_s, sem_sn, sem_w,
    ):
        wid = lax.axis_index("s") * num_cores + lax.axis_index("c")
        base = wid * b_per_w
        sl = pl.ds(base, b_per_w)

        pltpu.sync_copy(idx_hbm.at[sl], idx_v)

        cp_s = pltpu.async_copy(s_hbm.at[idx_v], s_v, sem_s)
        cp_sn = pltpu.async_copy(sn_hbm.at[idx_v], sn_v, sem_sn)

        def issue(g, carry):
            gbase = pl.multiple_of(g * _LANES, _LANES)
            v = idx_v[pl.ds(gbase, _LANES)]
            sem = sem_w.at[g]
            for l in range(_LANES):
                e = pl.multiple_of((v[l] >> 7) << 7, 128)
                jl = pl.ds(gbase + l, 1)
                pltpu.async_copy(a_hbm.at[:, pl.ds(e, 128)], a_win.at[jl, :], sem)
                pltpu.async_copy(r_hbm.at[:, pl.ds(e, 128)], r_win.at[jl, :], sem)
                pltpu.async_copy(dw_hbm.at[:, pl.ds(e, 128)], dw_win.at[jl, :], sem)
            return carry

        lax.fori_loop(0, n_grp, issue, 0)

        def extract(g, carry):
            gbase = pl.multiple_of(g * _LANES, _LANES)
            gs = pl.ds(gbase, _LANES)
            sem = sem_w.at[g]
            for l in range(_LANES):
                jl = pl.ds(gbase + l, 1)
                pltpu.make_async_copy(a_hbm.at[:, pl.ds(0, 128)], a_win.at[jl, :], sem).wait()
                pltpu.make_async_copy(r_hbm.at[:, pl.ds(0, 128)], r_win.at[jl, :], sem).wait()
                pltpu.make_async_copy(dw_hbm.at[:, pl.ds(0, 128)], dw_win.at[jl, :], sem).wait()
            rows = lax.iota(jnp.int32, _LANES) + gbase
            cols = idx_v[gs] & 127
            a_out_v[gs] = plsc.load_gather(a_win, [rows, cols])
            r_out_v[gs] = plsc.load_gather(r_win, [rows, cols])
            dw_out_v[gs] = plsc.load_gather(dw_win, [rows, cols])
            return carry

        lax.fori_loop(0, n_grp, extract, 0)

        pltpu.sync_copy(a_out_v, a_out.at[sl])
        pltpu.sync_copy(r_out_v, r_out.at[sl])
        pltpu.sync_copy(dw_out_v, dw_out.at[sl])

        cp_s.wait()
        pltpu.sync_copy(s_v, s_out.at[sl])
        cp_sn.wait()
        pltpu.sync_copy(sn_v, sn_out.at[sl])

    return sample_kernel


def kernel(s_buf, a_buf, r_buf, s_next_buf, dw_buf, ind):
    M, D = s_buf.shape
    B = ind.shape[0]
    fn = _make_sample_kernel(M, D, B)
    s, a, r, s_next, dw = fn(
        s_buf, a_buf.T, r_buf.T, s_next_buf, dw_buf.T, ind
    )
    return (s, a.reshape(B, 1), r.reshape(B, 1), s_next, dw.reshape(B, 1))


# speedup vs baseline: 6.2300x; 1.0146x over previous
import functools

import jax
import jax.numpy as jnp
from jax import lax
from jax.experimental import pallas as pl
from jax.experimental.pallas import tpu as pltpu
from jax.experimental.pallas import tpu_sc as plsc

_LANES = 16


@functools.lru_cache(maxsize=None)
def _make_sample_kernel(M, D, B):
    info = plsc.get_sparse_core_info()
    num_cores = info.num_cores
    num_subcores = info.num_subcores
    nw = num_cores * num_subcores
    b_per_w = B // nw
    assert b_per_w * nw == B and b_per_w % _LANES == 0
    n_grp = b_per_w // _LANES

    mesh = plsc.VectorSubcoreMesh(core_axis_name="c", subcore_axis_name="s")

    @functools.partial(
        pl.kernel,
        mesh=mesh,
        compiler_params=pltpu.CompilerParams(
            needs_layout_passes=False,
            skip_device_barrier=True,
            disable_bounds_checks=True,
            disable_semaphore_checks=True,
        ),
        out_type=(
            jax.ShapeDtypeStruct((B, D), jnp.float32),
            jax.ShapeDtypeStruct((B,), jnp.int32),
            jax.ShapeDtypeStruct((B,), jnp.float32),
            jax.ShapeDtypeStruct((B, D), jnp.float32),
            jax.ShapeDtypeStruct((B,), jnp.int32),
        ),
        scratch_types=[
            pltpu.VMEM((b_per_w,), jnp.int32),
            pltpu.VMEM((b_per_w, D), jnp.float32),
            pltpu.VMEM((b_per_w, D), jnp.float32),
            pltpu.VMEM((b_per_w, 128), jnp.int32),
            pltpu.VMEM((b_per_w, 128), jnp.float32),
            pltpu.VMEM((b_per_w, 128), jnp.int32),
            pltpu.VMEM((b_per_w,), jnp.int32),
            pltpu.VMEM((b_per_w,), jnp.float32),
            pltpu.VMEM((b_per_w,), jnp.int32),
            pltpu.SemaphoreType.DMA,
            pltpu.SemaphoreType.DMA,
            pltpu.SemaphoreType.DMA((8,)),
        ],
    )
    def sample_kernel(
        s_hbm, a_hbm, r_hbm, sn_hbm, dw_hbm, idx_hbm,
        s_out, a_out, r_out, sn_out, dw_out,
        idx_v, s_v, sn_v, a_win, r_win, dw_win,
        a_out_v, r_out_v, dw_out_v,
        sem_s, sem_sn, sem_w,
    ):
        wid = lax.axis_index("s") * num_cores + lax.axis_index("c")
        base = wid * b_per_w
        sl = pl.ds(base, b_per_w)

        pltpu.sync_copy(idx_hbm.at[sl], idx_v)

        def issue(g, carry):
            gbase = pl.multiple_of(g * _LANES, _LANES)
            v = idx_v[pl.ds(gbase, _LANES)]
            sem = sem_w.at[g]
            for l in range(_LANES):
                e = pl.multiple_of((v[l] >> 7) << 7, 128)
                jl = pl.ds(gbase + l, 1)
                pltpu.async_copy(a_hbm.at[:, pl.ds(e, 128)], a_win.at[jl, :], sem)
                pltpu.async_copy(r_hbm.at[:, pl.ds(e, 128)], r_win.at[jl, :], sem)
                pltpu.async_copy(dw_hbm.at[:, pl.ds(e, 128)], dw_win.at[jl, :], sem)
            return carry

        lax.fori_loop(0, n_grp, issue, 0)

        cp_s = pltpu.async_copy(s_hbm.at[idx_v], s_v, sem_s)
        cp_sn = pltpu.async_copy(sn_hbm.at[idx_v], sn_v, sem_sn)

        def extract(g, carry):
            gbase = pl.multiple_of(g * _LANES, _LANES)
            gs = pl.ds(gbase, _LANES)
            sem = sem_w.at[g]
            for l in range(_LANES):
                jl = pl.ds(gbase + l, 1)
                pltpu.make_async_copy(a_hbm.at[:, pl.ds(0, 128)], a_win.at[jl, :], sem).wait()
                pltpu.make_async_copy(r_hbm.at[:, pl.ds(0, 128)], r_win.at[jl, :], sem).wait()
                pltpu.make_async_copy(dw_hbm.at[:, pl.ds(0, 128)], dw_win.at[jl, :], sem).wait()
            rows = lax.iota(jnp.int32, _LANES) + gbase
            cols = idx_v[gs] & 127
            a_out_v[gs] = plsc.load_gather(a_win, [rows, cols])
            r_out_v[gs] = plsc.load_gather(r_win, [rows, cols])
            dw_out_v[gs] = plsc.load_gather(dw_win, [rows, cols])
            return carry

        lax.fori_loop(0, n_grp, extract, 0)

        pltpu.sync_copy(a_out_v, a_out.at[sl])
        pltpu.sync_copy(r_out_v, r_out.at[sl])
        pltpu.sync_copy(dw_out_v, dw_out.at[sl])

        cp_s.wait()
        pltpu.sync_copy(s_v, s_out.at[sl])
        cp_sn.wait()
        pltpu.sync_copy(sn_v, sn_out.at[sl])

    return sample_kernel


def kernel(s_buf, a_buf, r_buf, s_next_buf, dw_buf, ind):
    M, D = s_buf.shape
    B = ind.shape[0]
    fn = _make_sample_kernel(M, D, B)
    s, a, r, s_next, dw = fn(
        s_buf, a_buf.T, r_buf.T, s_next_buf, dw_buf.T, ind
    )
    return (s, a.reshape(B, 1), r.reshape(B, 1), s_next, dw.reshape(B, 1))
